# baseline (speedup 1.0000x reference)
import functools

import jax
import jax.numpy as jnp
from jax import lax
from jax.experimental import pallas as pl
from jax.experimental.pallas import tpu as pltpu
from jax.experimental.pallas import tpu_sc as plsc

N_NODES = 10000
N_PAD = 10112
E_EDGES = 320000
D = 64
K_STEPS = 10
ALPHA = 0.1

NC = 2
NS = 16
NW = NC * NS
EB = 128
NB = 80
E_PAD = NW * NB * EB
ROWS_PER_TILE = N_PAD // NS

_mesh = plsc.VectorSubcoreMesh(
    core_axis_name="c", subcore_axis_name="s", num_cores=NC, num_subcores=NS)


def _zero_rows(buf, nrows, width):
    z = jnp.zeros((16,), jnp.float32)

    def body(i, _):
        for j in range(width // 16):
            buf[i, pl.ds(j * 16, 16)] = z
        return 0

    lax.fori_loop(0, nrows, body, 0)


@functools.partial(
    pl.kernel,
    out_type=jax.ShapeDtypeStruct((NC, N_PAD, 16), jnp.float32),
    mesh=_mesh,
    compiler_params=pltpu.CompilerParams(use_tc_tiling_on_sc=False),
    scratch_types=[
        pltpu.VMEM((NB, EB), jnp.int32),
        pltpu.VMEM((NB, EB), jnp.int32),
        pltpu.VMEM((EB, 16), jnp.float32),
        pltpu.VMEM((EB, 16), jnp.float32),
        pltpu.VMEM((320, 16), jnp.float32),
        pltpu.VMEM_SHARED((N_PAD, 16), jnp.float32),
    ],
)
def _deg_kernel(src_hbm, dst_hbm, es_hbm, ed_hbm, deg_out,
                srcb, dstb, esb, edb, bounce, deg_sp):
    c = lax.axis_index("c")
    s = lax.axis_index("s")
    wid = c * NS + s

    pltpu.sync_copy(src_hbm.at[wid], srcb)
    pltpu.sync_copy(dst_hbm.at[wid], dstb)
    pltpu.sync_copy(es_hbm, esb)
    pltpu.sync_copy(ed_hbm, edb)

    base = s * ROWS_PER_TILE
    _zero_rows(bounce, 320, 16)
    pltpu.sync_copy(bounce, deg_sp.at[pl.ds(base, 320)])
    pltpu.sync_copy(bounce.at[pl.ds(0, 312)], deg_sp.at[pl.ds(base + 320, 312)])
    plsc.subcore_barrier()

    def body(b, _):
        pltpu.sync_copy(esb, deg_sp.at[srcb.at[b]], add=True)
        pltpu.sync_copy(edb, deg_sp.at[dstb.at[b]], add=True)
        return 0

    lax.fori_loop(0, NB, body, 0)
    plsc.subcore_barrier()

    pltpu.sync_copy(deg_sp.at[pl.ds(base, 320)], bounce)
    pltpu.sync_copy(bounce, deg_out.at[c, pl.ds(base, 320)])
    pltpu.sync_copy(deg_sp.at[pl.ds(base + 320, 312)], bounce.at[pl.ds(0, 312)])
    pltpu.sync_copy(bounce.at[pl.ds(0, 312)], deg_out.at[c, pl.ds(base + 320, 312)])


CH = 8
NCH = NB // CH
RING = 4


@functools.partial(
    pl.kernel,
    out_type=jax.ShapeDtypeStruct((NC, N_PAD, D), jnp.float32),
    mesh=_mesh,
    compiler_params=pltpu.CompilerParams(use_tc_tiling_on_sc=False),
    scratch_types=[
        pltpu.VMEM((3, CH, EB), jnp.int32),
        pltpu.VMEM((3, CH, EB), jnp.int32),
        pltpu.VMEM((RING, EB, D), jnp.float32),
        pltpu.VMEM_SHARED((N_PAD, D), jnp.float32),
        pltpu.VMEM_SHARED((N_PAD, D), jnp.float32),
    ] + [pltpu.SemaphoreType.DMA] * 10,
)
def _scatter_kernel(g_hbm, src_hbm, dst_hbm, agg_out,
                    srcc, dstc, rows, g_sp, agg_sp, *sems):
    c = lax.axis_index("c")
    s = lax.axis_index("s")
    wid = c * NS + s
    gsems, ssems, isem_s, isem_d = sems[:4], sems[4:8], sems[8], sems[9]

    base = s * ROWS_PER_TILE
    idx_h = [
        pltpu.async_copy(src_hbm.at[wid, pl.ds(0, CH)], srcc.at[0], gsems[0]),
        pltpu.async_copy(dst_hbm.at[wid, pl.ds(0, CH)], dstc.at[0], gsems[0]),
        pltpu.async_copy(src_hbm.at[wid, pl.ds(CH, CH)], srcc.at[1], isem_s),
        pltpu.async_copy(dst_hbm.at[wid, pl.ds(CH, CH)], dstc.at[1], isem_d),
    ]

    def _chunk(k):
        n = 128 if k < 4 else 120
        return pl.ds(base + 128 * k, n), n

    _zero_rows(rows.at[0], EB, D)
    zero_h = []
    for k in range(5):
        sl, n = _chunk(k)
        zero_h.append(pltpu.async_copy(
            rows.at[0].at[pl.ds(0, n)], agg_sp.at[sl], ssems[0]))

    gh = []
    for k in range(5):
        sl, _n = _chunk(k)
        gh.append(pltpu.async_copy(g_hbm.at[sl], g_sp.at[sl],
                                   gsems[1 + k % 3]))
    for h in gh:
        h.wait()
    for h in zero_h:
        h.wait()
    idx_h[0].wait()
    idx_h[1].wait()
    plsc.subcore_barrier()

    def _gather(buf, k, slot):
        pltpu.async_copy(g_sp.at[srcc.at[buf, k]], rows.at[slot], gsems[slot])

    def _scatter(buf, k, slot):
        pltpu.async_copy(rows.at[slot], agg_sp.at[dstc.at[buf, k]],
                         ssems[slot], add=True)

    def _wait(sem, slot):
        pltpu.make_async_copy(
            g_hbm.at[pl.ds(0, EB)], rows.at[slot], sem[slot]).wait()

    def _wait_idx():
        pltpu.make_async_copy(
            src_hbm.at[wid, pl.ds(0, CH)], srcc.at[0], isem_s).wait()
        pltpu.make_async_copy(
            dst_hbm.at[wid, pl.ds(0, CH)], dstc.at[0], isem_d).wait()

    for k in range(CH):
        slot = k % RING
        if k >= RING:
            _wait(ssems, slot)
        _gather(0, k, slot)
        if k >= 2:
            _wait(gsems, (k - 2) % RING)
            _scatter(0, k - 2, (k - 2) % RING)

    def body(ch, _):
        buf = ch % 3
        for k in (6, 7):
            _wait(gsems, k % RING)
            _scatter((ch - 1) % 3, k, k % RING)
        _wait_idx()

        @pl.when(ch < NCH - 1)
        def _():
            pltpu.async_copy(
                src_hbm.at[wid, pl.ds((ch + 1) * CH, CH)],
                srcc.at[(ch + 1) % 3], isem_s)
            pltpu.async_copy(
                dst_hbm.at[wid, pl.ds((ch + 1) * CH, CH)],
                dstc.at[(ch + 1) % 3], isem_d)

        for k in range(CH):
            slot = k % RING
            _wait(ssems, slot)
            _gather(buf, k, slot)
            if k >= 2:
                _wait(gsems, (k - 2) % RING)
                _scatter(buf, k - 2, (k - 2) % RING)
        return 0

    lax.fori_loop(1, NCH, body, 0)

    for k in (6, 7):
        _wait(gsems, k % RING)
        _scatter((NCH - 1) % 3, k, k % RING)
    for slot in range(RING):
        _wait(ssems, slot)
    plsc.subcore_barrier()

    wh = []
    for k in range(5):
        sl, _n = _chunk(k)
        wh.append(pltpu.async_copy(agg_sp.at[sl], agg_out.at[c, sl],
                                   ssems[k % 4]))
    for h in wh:
        h.wait()


_TC_R = 2528


def _mlp_body(x_ref, w1_ref, b1_ref, w2_ref, b2_ref, h_ref):
    h1 = jnp.maximum(
        jnp.dot(x_ref[...], w1_ref[...], preferred_element_type=jnp.float32)
        + b1_ref[...], 0.0)
    h_ref[...] = (jnp.dot(h1, w2_ref[...], preferred_element_type=jnp.float32)
                  + b2_ref[...])


def _mlp_call(x_pad, W1, b1, W2, b2):
    grid = N_PAD // _TC_R
    return pl.pallas_call(
        _mlp_body,
        grid=(grid,),
        in_specs=[
            pl.BlockSpec((_TC_R, 128), lambda i: (i, 0)),
            pl.BlockSpec((128, 128), lambda i: (0, 0)),
            pl.BlockSpec((1, 128), lambda i: (0, 0)),
            pl.BlockSpec((128, D), lambda i: (0, 0)),
            pl.BlockSpec((1, D), lambda i: (0, 0)),
        ],
        out_specs=pl.BlockSpec((_TC_R, D), lambda i: (i, 0)),
        out_shape=jax.ShapeDtypeStruct((N_PAD, D), jnp.float32),
    )(x_pad, W1, b1.reshape(1, 128), W2, b2.reshape(1, D))


def _coef_body(h_ref, deg_ref, g0_ref, cb_ref, sinv_ref):
    h = h_ref[...]
    ds = jnp.maximum(deg_ref[0, :, 0] + deg_ref[1, :, 0], 1.0)
    dd = jnp.maximum(deg_ref[0, :, 1] + deg_ref[1, :, 1], 1.0)
    nsrc = lax.rsqrt(ds)
    ndst = lax.rsqrt(dd)
    g0_ref[...] = h * nsrc[:, None]
    cb_ref[...] = jnp.broadcast_to(
        ((1.0 - ALPHA) * nsrc * ndst)[:, None], h.shape)
    sinv_ref[...] = jnp.broadcast_to(jnp.sqrt(ds)[:, None], h.shape)


def _coef_call(h, deg):
    grid = N_PAD // _TC_R
    out = jax.ShapeDtypeStruct((N_PAD, D), jnp.float32)
    return pl.pallas_call(
        _coef_body,
        grid=(grid,),
        in_specs=[
            pl.BlockSpec((_TC_R, D), lambda i: (i, 0)),
            pl.BlockSpec((NC, _TC_R, 16), lambda i: (0, i, 0)),
        ],
        out_specs=[pl.BlockSpec((_TC_R, D), lambda i: (i, 0))] * 3,
        out_shape=[out, out, out],
    )(h, deg)


def _blend_body(agg_ref, c_ref, base_ref, out_ref):
    out_ref[...] = (c_ref[...] * (agg_ref[0] + agg_ref[1])
                    + ALPHA * base_ref[...])


def _final_body(agg_ref, c_ref, base_ref, sinv_ref, out_ref):
    out_ref[...] = (c_ref[...] * (agg_ref[0] + agg_ref[1])
                    + ALPHA * base_ref[...]) * sinv_ref[...]


def _final_call(agg, coeff, base, sinv):
    grid = N_PAD // _TC_R
    return pl.pallas_call(
        _final_body,
        grid=(grid,),
        in_specs=[
            pl.BlockSpec((NC, _TC_R, D), lambda i: (0, i, 0)),
            pl.BlockSpec((_TC_R, D), lambda i: (i, 0)),
            pl.BlockSpec((_TC_R, D), lambda i: (i, 0)),
            pl.BlockSpec((_TC_R, D), lambda i: (i, 0)),
        ],
        out_specs=pl.BlockSpec((_TC_R, D), lambda i: (i, 0)),
        out_shape=jax.ShapeDtypeStruct((N_PAD, D), jnp.float32),
    )(agg, coeff, base, sinv)


def _blend_call(agg, coeff, base):
    grid = N_PAD // _TC_R
    return pl.pallas_call(
        _blend_body,
        grid=(grid,),
        in_specs=[
            pl.BlockSpec((NC, _TC_R, D), lambda i: (0, i, 0)),
            pl.BlockSpec((_TC_R, D), lambda i: (i, 0)),
            pl.BlockSpec((_TC_R, D), lambda i: (i, 0)),
        ],
        out_specs=pl.BlockSpec((_TC_R, D), lambda i: (i, 0)),
        out_shape=jax.ShapeDtypeStruct((N_PAD, D), jnp.float32),
    )(agg, coeff, base)


def kernel(x, edge_index, W1, b1, W2, b2):
    src = edge_index[0].astype(jnp.int32)
    dst = edge_index[1].astype(jnp.int32)
    pad = E_PAD - E_EDGES
    src = jnp.concatenate(
        [src, jnp.full((pad,), N_NODES, jnp.int32)]).reshape(NW, NB, EB)
    dst = jnp.concatenate(
        [dst, jnp.full((pad,), N_NODES, jnp.int32)]).reshape(NW, NB, EB)

    x_pad = jnp.pad(x, ((0, N_PAD - N_NODES), (0, 0)))
    col = jnp.arange(16, dtype=jnp.float32)
    e_src = jnp.broadcast_to((col == 0).astype(jnp.float32), (EB, 16))
    e_dst = jnp.broadcast_to((col == 1).astype(jnp.float32), (EB, 16))

    h = _mlp_call(x_pad, W1, b1, W2, b2)
    deg = _deg_kernel(src, dst, e_src, e_dst)
    g0, cb, sinv = _coef_call(h, deg)

    def step(g, _):
        agg = _scatter_kernel(g, src, dst)
        return _blend_call(agg, cb, g0), None

    g, _ = lax.scan(step, g0, None, length=K_STEPS - 1)
    agg = _scatter_kernel(g, src, dst)
    out = _final_call(agg, cb, g0, sinv)
    return out[:N_NODES]


# speedup vs baseline: 1.3070x; 1.3070x over previous
import functools

import jax
import jax.numpy as jnp
from jax import lax
from jax.experimental import pallas as pl
from jax.experimental.pallas import tpu as pltpu
from jax.experimental.pallas import tpu_sc as plsc

N_NODES = 10000
N_PAD = 10112
E_EDGES = 320000
D = 64
K_STEPS = 10
ALPHA = 0.1

NC = 2
NS = 16
NW = NC * NS
EB = 128
NB = 80
E_PAD = NW * NB * EB
ROWS_PER_TILE = N_PAD // NS

_mesh = plsc.VectorSubcoreMesh(
    core_axis_name="c", subcore_axis_name="s", num_cores=NC, num_subcores=NS)


def _zero_rows(buf, nrows, width):
    z = jnp.zeros((16,), jnp.float32)

    def body(i, _):
        for j in range(width // 16):
            buf[i, pl.ds(j * 16, 16)] = z
        return 0

    lax.fori_loop(0, nrows, body, 0)


def _zero_rows_bf16(buf, nrows, width):
    z = jnp.zeros((32,), jnp.bfloat16)

    def body(i, _):
        for j in range(width // 32):
            buf[i, pl.ds(j * 32, 32)] = z
        return 0

    lax.fori_loop(0, nrows, body, 0)


@functools.partial(
    pl.kernel,
    out_type=jax.ShapeDtypeStruct((NC, N_PAD, 16), jnp.float32),
    mesh=_mesh,
    compiler_params=pltpu.CompilerParams(use_tc_tiling_on_sc=False),
    scratch_types=[
        pltpu.VMEM((NB, EB), jnp.int32),
        pltpu.VMEM((NB, EB), jnp.int32),
        pltpu.VMEM((EB, 16), jnp.float32),
        pltpu.VMEM((EB, 16), jnp.float32),
        pltpu.VMEM((320, 16), jnp.float32),
        pltpu.VMEM_SHARED((N_PAD, 16), jnp.float32),
    ],
)
def _deg_kernel(src_hbm, dst_hbm, es_hbm, ed_hbm, deg_out,
                srcb, dstb, esb, edb, bounce, deg_sp):
    c = lax.axis_index("c")
    s = lax.axis_index("s")
    wid = c * NS + s

    pltpu.sync_copy(src_hbm.at[wid], srcb)
    pltpu.sync_copy(dst_hbm.at[wid], dstb)
    pltpu.sync_copy(es_hbm, esb)
    pltpu.sync_copy(ed_hbm, edb)

    base = s * ROWS_PER_TILE
    _zero_rows(bounce, 320, 16)
    pltpu.sync_copy(bounce, deg_sp.at[pl.ds(base, 320)])
    pltpu.sync_copy(bounce.at[pl.ds(0, 312)], deg_sp.at[pl.ds(base + 320, 312)])
    plsc.subcore_barrier()

    def body(b, _):
        pltpu.sync_copy(esb, deg_sp.at[srcb.at[b]], add=True)
        pltpu.sync_copy(edb, deg_sp.at[dstb.at[b]], add=True)
        return 0

    lax.fori_loop(0, NB, body, 0)
    plsc.subcore_barrier()

    pltpu.sync_copy(deg_sp.at[pl.ds(base, 320)], bounce)
    pltpu.sync_copy(bounce, deg_out.at[c, pl.ds(base, 320)])
    pltpu.sync_copy(deg_sp.at[pl.ds(base + 320, 312)], bounce.at[pl.ds(0, 312)])
    pltpu.sync_copy(bounce.at[pl.ds(0, 312)], deg_out.at[c, pl.ds(base + 320, 312)])


CH = 8
NCH = NB // CH
RING = 4


@functools.partial(
    pl.kernel,
    out_type=jax.ShapeDtypeStruct((NC, N_PAD, D), jnp.bfloat16),
    mesh=_mesh,
    compiler_params=pltpu.CompilerParams(use_tc_tiling_on_sc=False),
    scratch_types=[
        pltpu.VMEM((3, CH, EB), jnp.int32),
        pltpu.VMEM((3, CH, EB), jnp.int32),
        pltpu.VMEM((RING, EB, D), jnp.bfloat16),
        pltpu.VMEM_SHARED((N_PAD, D), jnp.bfloat16),
        pltpu.VMEM_SHARED((N_PAD, D), jnp.bfloat16),
    ] + [pltpu.SemaphoreType.DMA] * 10,
)
def _scatter_kernel(g_hbm, src_hbm, dst_hbm, agg_out,
                    srcc, dstc, rows, g_sp, agg_sp, *sems):
    c = lax.axis_index("c")
    s = lax.axis_index("s")
    wid = c * NS + s
    gsems, ssems, isem_s, isem_d = sems[:4], sems[4:8], sems[8], sems[9]

    base = s * ROWS_PER_TILE
    idx_h = [
        pltpu.async_copy(src_hbm.at[wid, pl.ds(0, CH)], srcc.at[0], gsems[0]),
        pltpu.async_copy(dst_hbm.at[wid, pl.ds(0, CH)], dstc.at[0], gsems[0]),
        pltpu.async_copy(src_hbm.at[wid, pl.ds(CH, CH)], srcc.at[1], isem_s),
        pltpu.async_copy(dst_hbm.at[wid, pl.ds(CH, CH)], dstc.at[1], isem_d),
    ]

    def _chunk(k):
        n = 128 if k < 4 else 120
        return pl.ds(base + 128 * k, n), n

    _zero_rows_bf16(rows.at[0], EB, D)
    zero_h = []
    for k in range(5):
        sl, n = _chunk(k)
        zero_h.append(pltpu.async_copy(
            rows.at[0].at[pl.ds(0, n)], agg_sp.at[sl], ssems[0]))

    gh = []
    for k in range(5):
        sl, _n = _chunk(k)
        gh.append(pltpu.async_copy(g_hbm.at[sl], g_sp.at[sl],
                                   gsems[1 + k % 3]))
    for h in gh:
        h.wait()
    for h in zero_h:
        h.wait()
    idx_h[0].wait()
    idx_h[1].wait()
    plsc.subcore_barrier()

    def _gather(buf, k, slot):
        pltpu.async_copy(g_sp.at[srcc.at[buf, k]], rows.at[slot], gsems[slot])

    def _scatter(buf, k, slot):
        pltpu.async_copy(rows.at[slot], agg_sp.at[dstc.at[buf, k]],
                         ssems[slot], add=True)

    def _wait(sem, slot):
        pltpu.make_async_copy(
            g_hbm.at[pl.ds(0, EB)], rows.at[slot], sem[slot]).wait()

    def _wait_idx():
        pltpu.make_async_copy(
            src_hbm.at[wid, pl.ds(0, CH)], srcc.at[0], isem_s).wait()
        pltpu.make_async_copy(
            dst_hbm.at[wid, pl.ds(0, CH)], dstc.at[0], isem_d).wait()

    for k in range(CH):
        slot = k % RING
        if k >= RING:
            _wait(ssems, slot)
        _gather(0, k, slot)
        if k >= 2:
            _wait(gsems, (k - 2) % RING)
            _scatter(0, k - 2, (k - 2) % RING)

    def body(ch, _):
        buf = ch % 3
        for k in (6, 7):
            _wait(gsems, k % RING)
            _scatter((ch - 1) % 3, k, k % RING)
        _wait_idx()

        @pl.when(ch < NCH - 1)
        def _():
            pltpu.async_copy(
                src_hbm.at[wid, pl.ds((ch + 1) * CH, CH)],
                srcc.at[(ch + 1) % 3], isem_s)
            pltpu.async_copy(
                dst_hbm.at[wid, pl.ds((ch + 1) * CH, CH)],
                dstc.at[(ch + 1) % 3], isem_d)

        for k in range(CH):
            slot = k % RING
            _wait(ssems, slot)
            _gather(buf, k, slot)
            if k >= 2:
                _wait(gsems, (k - 2) % RING)
                _scatter(buf, k - 2, (k - 2) % RING)
        return 0

    lax.fori_loop(1, NCH, body, 0)

    for k in (6, 7):
        _wait(gsems, k % RING)
        _scatter((NCH - 1) % 3, k, k % RING)
    for slot in range(RING):
        _wait(ssems, slot)
    plsc.subcore_barrier()

    wh = []
    for k in range(5):
        sl, _n = _chunk(k)
        wh.append(pltpu.async_copy(agg_sp.at[sl], agg_out.at[c, sl],
                                   ssems[k % 4]))
    for h in wh:
        h.wait()


_TC_R = 2528


def _mlp_body(x_ref, w1_ref, b1_ref, w2_ref, b2_ref, h_ref):
    h1 = jnp.maximum(
        jnp.dot(x_ref[...], w1_ref[...], preferred_element_type=jnp.float32)
        + b1_ref[...], 0.0)
    h_ref[...] = (jnp.dot(h1, w2_ref[...], preferred_element_type=jnp.float32)
                  + b2_ref[...])


def _mlp_call(x_pad, W1, b1, W2, b2):
    grid = N_PAD // _TC_R
    return pl.pallas_call(
        _mlp_body,
        grid=(grid,),
        in_specs=[
            pl.BlockSpec((_TC_R, 128), lambda i: (i, 0)),
            pl.BlockSpec((128, 128), lambda i: (0, 0)),
            pl.BlockSpec((1, 128), lambda i: (0, 0)),
            pl.BlockSpec((128, D), lambda i: (0, 0)),
            pl.BlockSpec((1, D), lambda i: (0, 0)),
        ],
        out_specs=pl.BlockSpec((_TC_R, D), lambda i: (i, 0)),
        out_shape=jax.ShapeDtypeStruct((N_PAD, D), jnp.float32),
    )(x_pad, W1, b1.reshape(1, 128), W2, b2.reshape(1, D))


def _coef_body(h_ref, deg_ref, g0_ref, cb_ref, sinv_ref):
    h = h_ref[...]
    ds = jnp.maximum(deg_ref[0, :, 0] + deg_ref[1, :, 0], 1.0)
    dd = jnp.maximum(deg_ref[0, :, 1] + deg_ref[1, :, 1], 1.0)
    nsrc = lax.rsqrt(ds)
    ndst = lax.rsqrt(dd)
    g0_ref[...] = h * nsrc[:, None]
    cb_ref[...] = jnp.broadcast_to(
        ((1.0 - ALPHA) * nsrc * ndst)[:, None], h.shape)
    sinv_ref[...] = jnp.broadcast_to(jnp.sqrt(ds)[:, None], h.shape)


def _coef_call(h, deg):
    grid = N_PAD // _TC_R
    out = jax.ShapeDtypeStruct((N_PAD, D), jnp.float32)
    return pl.pallas_call(
        _coef_body,
        grid=(grid,),
        in_specs=[
            pl.BlockSpec((_TC_R, D), lambda i: (i, 0)),
            pl.BlockSpec((NC, _TC_R, 16), lambda i: (0, i, 0)),
        ],
        out_specs=[pl.BlockSpec((_TC_R, D), lambda i: (i, 0))] * 3,
        out_shape=[out, out, out],
    )(h, deg)


def _blend_body(agg_ref, c_ref, base_ref, out_ref):
    agg = (agg_ref[0].astype(jnp.float32) + agg_ref[1].astype(jnp.float32))
    out_ref[...] = (c_ref[...] * agg
                    + ALPHA * base_ref[...]).astype(jnp.bfloat16)


def _final_body(agg_ref, c_ref, base_ref, sinv_ref, out_ref):
    agg = (agg_ref[0].astype(jnp.float32) + agg_ref[1].astype(jnp.float32))
    out_ref[...] = (c_ref[...] * agg
                    + ALPHA * base_ref[...]) * sinv_ref[...]


def _final_call(agg, coeff, base, sinv):
    grid = N_PAD // _TC_R
    return pl.pallas_call(
        _final_body,
        grid=(grid,),
        in_specs=[
            pl.BlockSpec((NC, _TC_R, D), lambda i: (0, i, 0)),
            pl.BlockSpec((_TC_R, D), lambda i: (i, 0)),
            pl.BlockSpec((_TC_R, D), lambda i: (i, 0)),
            pl.BlockSpec((_TC_R, D), lambda i: (i, 0)),
        ],
        out_specs=pl.BlockSpec((_TC_R, D), lambda i: (i, 0)),
        out_shape=jax.ShapeDtypeStruct((N_PAD, D), jnp.float32),
    )(agg, coeff, base, sinv)


def _blend_call(agg, coeff, base):
    grid = N_PAD // _TC_R
    return pl.pallas_call(
        _blend_body,
        grid=(grid,),
        in_specs=[
            pl.BlockSpec((NC, _TC_R, D), lambda i: (0, i, 0)),
            pl.BlockSpec((_TC_R, D), lambda i: (i, 0)),
            pl.BlockSpec((_TC_R, D), lambda i: (i, 0)),
        ],
        out_specs=pl.BlockSpec((_TC_R, D), lambda i: (i, 0)),
        out_shape=jax.ShapeDtypeStruct((N_PAD, D), jnp.bfloat16),
    )(agg, coeff, base)


def kernel(x, edge_index, W1, b1, W2, b2):
    src = edge_index[0].astype(jnp.int32)
    dst = edge_index[1].astype(jnp.int32)
    pad = E_PAD - E_EDGES
    src = jnp.concatenate(
        [src, jnp.full((pad,), N_NODES, jnp.int32)]).reshape(NW, NB, EB)
    dst = jnp.concatenate(
        [dst, jnp.full((pad,), N_NODES, jnp.int32)]).reshape(NW, NB, EB)

    x_pad = jnp.pad(x, ((0, N_PAD - N_NODES), (0, 0)))
    col = jnp.arange(16, dtype=jnp.float32)
    e_src = jnp.broadcast_to((col == 0).astype(jnp.float32), (EB, 16))
    e_dst = jnp.broadcast_to((col == 1).astype(jnp.float32), (EB, 16))

    h = _mlp_call(x_pad, W1, b1, W2, b2)
    deg = _deg_kernel(src, dst, e_src, e_dst)
    g0, cb, sinv = _coef_call(h, deg)

    def step(g, _):
        agg = _scatter_kernel(g, src, dst)
        return _blend_call(agg, cb, g0), None

    g, _ = lax.scan(step, g0.astype(jnp.bfloat16), None, length=K_STEPS - 1)
    agg = _scatter_kernel(g, src, dst)
    out = _final_call(agg, cb, g0, sinv)
    return out[:N_NODES]
